# baseline (speedup 1.0000x reference)
import functools

import jax
import jax.numpy as jnp
from jax import lax
from jax.experimental import pallas as pl
from jax.experimental.pallas import tpu as pltpu
from jax.experimental.pallas import tpu_sc as plsc

N = 4096
D = 768
E = 64
NP = 2 * N
TB = 512
RB = 512
T = 128
NT = 128
R = NT * T
NW = 32
DCH = 128
TCH = 32

_F32 = jnp.float32
_I32 = jnp.int32


D2 = D // 2


def _pack_cols(v):
    b = lax.bitcast_convert_type(v.astype(jnp.bfloat16), jnp.uint16)
    lo = b[:, :D2].astype(jnp.uint32)
    hi = b[:, D2:].astype(jnp.uint32)
    return lax.bitcast_convert_type(lo | (hi << 16), _F32)


def _unpack_cols(p):
    u = lax.bitcast_convert_type(p, jnp.uint32)
    lo = lax.bitcast_convert_type(u << 16, _F32)
    hi = lax.bitcast_convert_type(u & jnp.uint32(0xFFFF0000), _F32)
    return jnp.concatenate([lo, hi], axis=1)


def _gate_body(x_ref, wg_ref, bg_ref, be_ref, i_ref, sx0_ref, sx1_ref, bias_ref):
    x = x_ref[...]
    logits = lax.dot_general(
        x, wg_ref[...], (((1,), (1,)), ((), ())),
        preferred_element_type=_F32) + bg_ref[...]
    m = jnp.max(logits, axis=1, keepdims=True)
    ex = jnp.exp(logits - m)
    p = ex / jnp.sum(ex, axis=1, keepdims=True)
    iota = lax.broadcasted_iota(_I32, (TB, E), 1)
    m1 = jnp.max(p, axis=1, keepdims=True)
    i1 = jnp.min(jnp.where(p == m1, iota, E), axis=1, keepdims=True)
    oh1 = iota == i1
    pm = jnp.where(oh1, -1.0, p)
    m2 = jnp.max(pm, axis=1, keepdims=True)
    i2 = jnp.min(jnp.where(pm == m2, iota, E), axis=1, keepdims=True)
    oh2 = iota == i2
    i_ref[...] = jnp.concatenate([i1, i2], axis=1)
    sx0_ref[...] = _pack_cols(m1 * x)
    sx1_ref[...] = _pack_cols(m2 * x)
    combine = jnp.where(oh1, m1, 0.0) + jnp.where(oh2, m2, 0.0)
    bias_ref[...] = lax.dot_general(
        combine, be_ref[...], (((1,), (0,)), ((), ())),
        preferred_element_type=_F32)


def _gate(x, Wg, bg2, be, interpret=False):
    return pl.pallas_call(
        _gate_body,
        grid=(N // TB,),
        in_specs=[
            pl.BlockSpec((TB, D), lambda i: (i, 0)),
            pl.BlockSpec((E, D), lambda i: (0, 0)),
            pl.BlockSpec((1, E), lambda i: (0, 0)),
            pl.BlockSpec((E, D), lambda i: (0, 0)),
        ],
        out_specs=[
            pl.BlockSpec((TB, 2), lambda i: (i, 0)),
            pl.BlockSpec((TB, D2), lambda i: (i, 0)),
            pl.BlockSpec((TB, D2), lambda i: (i, 0)),
            pl.BlockSpec((TB, D), lambda i: (i, 0)),
        ],
        out_shape=[
            jax.ShapeDtypeStruct((N, 2), _I32),
            jax.ShapeDtypeStruct((N, D2), _F32),
            jax.ShapeDtypeStruct((N, D2), _F32),
            jax.ShapeDtypeStruct((N, D), _F32),
        ],
        interpret=interpret,
    )(x, Wg, bg2, be)


def _route_body(i_ref, dflat_ref, gid_ref, r1_ref, r2_ref):
    nblk = N // RB
    ti = lax.broadcasted_iota(_I32, (RB, RB), 0)
    tj = lax.broadcasted_iota(_I32, (RB, RB), 1)
    tl = (ti > tj).astype(_F32)
    iota_b = lax.broadcasted_iota(_I32, (RB, E), 1)

    def blk(b, carry):
        c0, c1 = carry
        i1b = i_ref[pl.ds(b * RB, RB), pl.ds(0, 1)]
        i2b = i_ref[pl.ds(b * RB, RB), pl.ds(1, 1)]
        h1 = (iota_b == i1b).astype(_F32)
        h2 = (iota_b == i2b).astype(_F32)
        r1_ref[pl.ds(b * RB, RB), :] = lax.dot_general(
            tl, h1, (((1,), (0,)), ((), ())), preferred_element_type=_F32) + c0
        r2_ref[pl.ds(b * RB, RB), :] = lax.dot_general(
            tl, h2, (((1,), (0,)), ((), ())), preferred_element_type=_F32) + c1
        return (c0 + jnp.sum(h1, axis=0, keepdims=True),
                c1 + jnp.sum(h2, axis=0, keepdims=True))

    c0, c1 = lax.fori_loop(
        0, nblk, blk,
        (jnp.zeros((1, E), _F32), jnp.zeros((1, E), _F32)))
    counts = c0 + c1
    ptiles = jnp.floor((counts + float(T - 1)) * (1.0 / T))
    ui = lax.broadcasted_iota(_I32, (E, E), 0)
    uj = lax.broadcasted_iota(_I32, (E, E), 1)
    ul = (ui < uj).astype(_F32)
    tile_off = lax.dot_general(
        ptiles, ul, (((1,), (0,)), ((), ())), preferred_element_type=_F32)
    off_pad = tile_off * float(T)

    iota_f = lax.broadcasted_iota(_I32, (N, E), 1)
    h1f = (iota_f == i_ref[:, pl.ds(0, 1)]).astype(_F32)
    h2f = (iota_f == i_ref[:, pl.ds(1, 1)]).astype(_F32)
    d0 = jnp.sum(h1f * (r1_ref[...] + off_pad), axis=1, keepdims=True)
    d1 = jnp.sum(h2f * (r2_ref[...] + off_pad + c0), axis=1, keepdims=True)
    dflat_ref[pl.ds(0, N), :] = d0.astype(_I32)
    dflat_ref[pl.ds(N, N), :] = d1.astype(_I32)

    gid_ref[pl.ds(0, 1), :] = tile_off.astype(_I32)
    gid_ref[pl.ds(1, 1), :] = ptiles.astype(_I32)


def _route(idx2, interpret=False):
    return pl.pallas_call(
        _route_body,
        out_shape=[
            jax.ShapeDtypeStruct((NP, 1), _I32),
            jax.ShapeDtypeStruct((2, E), _I32),
        ],
        scratch_shapes=[
            pltpu.VMEM((N, E), _F32),
            pltpu.VMEM((N, E), _F32),
        ],
        interpret=interpret,
    )(idx2)


def _dispatch_body(sx0, sx1, dflat, xs_out, xbuf0, xbuf1, dbuf0, dbuf1, s0, s1):
    wid = lax.axis_index("s") * 2 + lax.axis_index("c")
    per = NP // NW
    base = wid * per

    def _do(src, row0):
        pltpu.sync_copy(src.at[pl.ds(row0, DCH)], xbuf0)
        pltpu.sync_copy(dflat.at[pl.ds(base, DCH)], dbuf0)
        sc0 = pltpu.async_copy(xbuf0, xs_out.at[dbuf0], s0)
        pltpu.sync_copy(src.at[pl.ds(row0 + DCH, DCH)], xbuf1)
        pltpu.sync_copy(dflat.at[pl.ds(base + DCH, DCH)], dbuf1)
        sc1 = pltpu.async_copy(xbuf1, xs_out.at[dbuf1], s1)
        sc0.wait()
        sc1.wait()

    @pl.when(wid < NW // 2)
    def _():
        _do(sx0, base)

    @pl.when(wid >= NW // 2)
    def _():
        _do(sx1, base - N)


def _dispatch(sx0, sx1, dflat):
    fn = functools.partial(
        pl.kernel,
        mesh=plsc.VectorSubcoreMesh(core_axis_name="c", subcore_axis_name="s"),
        out_type=jax.ShapeDtypeStruct((R, D2), _F32),
        scratch_types=[
            pltpu.VMEM((DCH, D2), _F32),
            pltpu.VMEM((DCH, D2), _F32),
            pltpu.VMEM((DCH,), _I32),
            pltpu.VMEM((DCH,), _I32),
            pltpu.SemaphoreType.DMA,
            pltpu.SemaphoreType.DMA,
        ],
    )(_dispatch_body)
    return fn(sx0, sx1, dflat)


def _mm_body(meta_ref, xs_ref, we_ref, yw_ref, wbuf, ybuf, sem_w, sem_y):
    def wcopy(e, slot):
        return pltpu.make_async_copy(we_ref.at[e], wbuf.at[slot], sem_w.at[slot])

    wcopy(0, 0).start()

    def expert(e, g):
        slot = lax.rem(e, 2)

        @pl.when(e + 1 < E)
        def _():
            wcopy(e + 1, lax.rem(e + 1, 2)).start()

        wcopy(e, slot).wait()
        base = meta_ref[0, e] * T
        nt = meta_ref[1, e]

        def tile(i, gi):
            ys = lax.rem(gi, 2)
            row = base + i * T

            def ycopy():
                return pltpu.make_async_copy(
                    ybuf.at[ys], yw_ref.at[pl.ds(row, T)], sem_y.at[ys])

            @pl.when(gi >= 2)
            def _():
                ycopy().wait()

            xb = _unpack_cols(xs_ref[pl.ds(row, T), :]).astype(jnp.bfloat16)
            ybuf[ys, ...] = lax.dot_general(
                xb, wbuf[slot],
                (((1,), (1,)), ((), ())), preferred_element_type=_F32)
            ycopy().start()
            return gi + 1

        return lax.fori_loop(0, nt, tile, g)

    lax.fori_loop(0, E, expert, 0)
    for ys in range(2):
        pltpu.make_async_copy(
            ybuf.at[ys], yw_ref.at[pl.ds(0, T)], sem_y.at[ys]).wait()


def _mm(meta, xs, We, interpret=False):
    return pl.pallas_call(
        _mm_body,
        in_specs=[
            pl.BlockSpec(memory_space=pltpu.SMEM),
            pl.BlockSpec(memory_space=pltpu.VMEM),
            pl.BlockSpec(memory_space=pl.ANY),
        ],
        out_specs=pl.BlockSpec(memory_space=pl.ANY),
        out_shape=jax.ShapeDtypeStruct((R, D), _F32),
        scratch_shapes=[
            pltpu.VMEM((2, D, D), jnp.bfloat16),
            pltpu.VMEM((2, T, D), _F32),
            pltpu.SemaphoreType.DMA((2,)),
            pltpu.SemaphoreType.DMA((2,)),
        ],
        interpret=interpret,
    )(meta, xs, We)


def _combine_body(bias, yw, dflat, out_hbm, acc, g0, g1, db0, db1, s0, s1):
    wid = lax.axis_index("s") * 2 + lax.axis_index("c")
    tpw = N // NW

    def addrows(gbuf):
        def addrow(j, _):
            for k in range(D // 16):
                sl = pl.ds(k * 16, 16)
                acc[j, sl] = acc[j, sl] + gbuf[j, sl]
            return 0

        lax.fori_loop(0, TCH, addrow, 0)

    for c in range(tpw // TCH):
        tb = wid * tpw + c * TCH
        pltpu.sync_copy(dflat.at[pl.ds(tb, TCH)], db0)
        pltpu.sync_copy(dflat.at[pl.ds(N + tb, TCH)], db1)
        cp0 = pltpu.async_copy(yw.at[db0], g0, s0)
        cp1 = pltpu.async_copy(yw.at[db1], g1, s1)
        pltpu.sync_copy(bias.at[pl.ds(tb, TCH)], acc)
        cp0.wait()
        addrows(g0)
        cp1.wait()
        addrows(g1)
        pltpu.sync_copy(acc, out_hbm.at[pl.ds(tb, TCH)])


def _combine(bias, yw, dflat):
    fn = functools.partial(
        pl.kernel,
        mesh=plsc.VectorSubcoreMesh(core_axis_name="c", subcore_axis_name="s"),
        out_type=jax.ShapeDtypeStruct((N, D), _F32),
        scratch_types=[
            pltpu.VMEM((TCH, D), _F32),
            pltpu.VMEM((TCH, D), _F32),
            pltpu.VMEM((TCH, D), _F32),
            pltpu.VMEM((TCH,), _I32),
            pltpu.VMEM((TCH,), _I32),
            pltpu.SemaphoreType.DMA,
            pltpu.SemaphoreType.DMA,
        ],
    )(_combine_body)
    return fn(bias, yw, dflat)


def kernel(x, Wg, bg, We, be):
    bg2 = bg.reshape(1, E)
    idx2, sx0, sx1, bias = _gate(x, Wg, bg2, be)
    dflat2, meta = _route(idx2)
    dflat = dflat2.reshape(NP)
    xs = _dispatch(sx0, sx1, dflat)
    yw = _mm(meta, xs, We.astype(jnp.bfloat16))
    return _combine(bias, yw, dflat)


# speedup vs baseline: 1.2495x; 1.2495x over previous
import functools

import jax
import jax.numpy as jnp
from jax import lax
from jax.experimental import pallas as pl
from jax.experimental.pallas import tpu as pltpu
from jax.experimental.pallas import tpu_sc as plsc

N = 4096
D = 768
E = 64
NP = 2 * N
TB = 512
RB = 512
T = 128
NT = 128
R = NT * T
NW = 32
DCH = 128
TCH = 32

_F32 = jnp.float32
_I32 = jnp.int32


D2 = D // 2


def _pack_cols(v):
    b = lax.bitcast_convert_type(v.astype(jnp.bfloat16), jnp.uint16)
    lo = b[:, :D2].astype(jnp.uint32)
    hi = b[:, D2:].astype(jnp.uint32)
    return lax.bitcast_convert_type(lo | (hi << 16), _F32)


def _unpack_cols(p):
    u = lax.bitcast_convert_type(p, jnp.uint32)
    lo = lax.bitcast_convert_type(u << 16, _F32)
    hi = lax.bitcast_convert_type(u & jnp.uint32(0xFFFF0000), _F32)
    return jnp.concatenate([lo, hi], axis=1)


def _gate_body(x_ref, wg_ref, bg_ref, be_ref, i_ref, sx0_ref, sx1_ref, bias_ref):
    x = x_ref[...]
    logits = lax.dot_general(
        x, wg_ref[...], (((1,), (1,)), ((), ())),
        preferred_element_type=_F32) + bg_ref[...]
    m = jnp.max(logits, axis=1, keepdims=True)
    ex = jnp.exp(logits - m)
    p = ex / jnp.sum(ex, axis=1, keepdims=True)
    iota = lax.broadcasted_iota(_I32, (TB, E), 1)
    m1 = jnp.max(p, axis=1, keepdims=True)
    i1 = jnp.min(jnp.where(p == m1, iota, E), axis=1, keepdims=True)
    oh1 = iota == i1
    pm = jnp.where(oh1, -1.0, p)
    m2 = jnp.max(pm, axis=1, keepdims=True)
    i2 = jnp.min(jnp.where(pm == m2, iota, E), axis=1, keepdims=True)
    oh2 = iota == i2
    i_ref[...] = jnp.concatenate([i1, i2], axis=1)
    sx0_ref[...] = _pack_cols(m1 * x)
    sx1_ref[...] = _pack_cols(m2 * x)
    combine = jnp.where(oh1, m1, 0.0) + jnp.where(oh2, m2, 0.0)
    bias_ref[...] = lax.dot_general(
        combine, be_ref[...], (((1,), (0,)), ((), ())),
        preferred_element_type=_F32)


def _gate(x, Wg, bg2, be, interpret=False):
    return pl.pallas_call(
        _gate_body,
        grid=(N // TB,),
        in_specs=[
            pl.BlockSpec((TB, D), lambda i: (i, 0)),
            pl.BlockSpec((E, D), lambda i: (0, 0)),
            pl.BlockSpec((1, E), lambda i: (0, 0)),
            pl.BlockSpec((E, D), lambda i: (0, 0)),
        ],
        out_specs=[
            pl.BlockSpec((TB, 2), lambda i: (i, 0)),
            pl.BlockSpec((TB, D2), lambda i: (i, 0)),
            pl.BlockSpec((TB, D2), lambda i: (i, 0)),
            pl.BlockSpec((TB, D), lambda i: (i, 0)),
        ],
        out_shape=[
            jax.ShapeDtypeStruct((N, 2), _I32),
            jax.ShapeDtypeStruct((N, D2), _F32),
            jax.ShapeDtypeStruct((N, D2), _F32),
            jax.ShapeDtypeStruct((N, D), _F32),
        ],
        interpret=interpret,
    )(x, Wg, bg2, be)


def _route_body(i_ref, dflat_ref, gid_ref, r1_ref, r2_ref):
    nblk = N // RB
    ti = lax.broadcasted_iota(_I32, (RB, RB), 0)
    tj = lax.broadcasted_iota(_I32, (RB, RB), 1)
    tl = (ti > tj).astype(_F32)
    iota_b = lax.broadcasted_iota(_I32, (RB, E), 1)

    def blk(b, carry):
        c0, c1 = carry
        i1b = i_ref[pl.ds(b * RB, RB), pl.ds(0, 1)]
        i2b = i_ref[pl.ds(b * RB, RB), pl.ds(1, 1)]
        h1 = (iota_b == i1b).astype(_F32)
        h2 = (iota_b == i2b).astype(_F32)
        r1_ref[pl.ds(b * RB, RB), :] = lax.dot_general(
            tl, h1, (((1,), (0,)), ((), ())), preferred_element_type=_F32) + c0
        r2_ref[pl.ds(b * RB, RB), :] = lax.dot_general(
            tl, h2, (((1,), (0,)), ((), ())), preferred_element_type=_F32) + c1
        return (c0 + jnp.sum(h1, axis=0, keepdims=True),
                c1 + jnp.sum(h2, axis=0, keepdims=True))

    c0, c1 = lax.fori_loop(
        0, nblk, blk,
        (jnp.zeros((1, E), _F32), jnp.zeros((1, E), _F32)))
    counts = c0 + c1
    ptiles = jnp.floor((counts + float(T - 1)) * (1.0 / T))
    ui = lax.broadcasted_iota(_I32, (E, E), 0)
    uj = lax.broadcasted_iota(_I32, (E, E), 1)
    ul = (ui < uj).astype(_F32)
    tile_off = lax.dot_general(
        ptiles, ul, (((1,), (0,)), ((), ())), preferred_element_type=_F32)
    off_pad = tile_off * float(T)

    iota_f = lax.broadcasted_iota(_I32, (N, E), 1)
    h1f = (iota_f == i_ref[:, pl.ds(0, 1)]).astype(_F32)
    h2f = (iota_f == i_ref[:, pl.ds(1, 1)]).astype(_F32)
    d0 = jnp.sum(h1f * (r1_ref[...] + off_pad), axis=1, keepdims=True)
    d1 = jnp.sum(h2f * (r2_ref[...] + off_pad + c0), axis=1, keepdims=True)
    dflat_ref[pl.ds(0, N), :] = d0.astype(_I32)
    dflat_ref[pl.ds(N, N), :] = d1.astype(_I32)

    gid_ref[pl.ds(0, 1), :] = tile_off.astype(_I32)
    gid_ref[pl.ds(1, 1), :] = ptiles.astype(_I32)


def _route(idx2, interpret=False):
    return pl.pallas_call(
        _route_body,
        out_shape=[
            jax.ShapeDtypeStruct((NP, 1), _I32),
            jax.ShapeDtypeStruct((2, E), _I32),
        ],
        scratch_shapes=[
            pltpu.VMEM((N, E), _F32),
            pltpu.VMEM((N, E), _F32),
        ],
        interpret=interpret,
    )(idx2)


def _dispatch_body(sx0, sx1, dflat, xs_out, xbuf0, xbuf1, dbuf0, dbuf1, s0, s1):
    wid = lax.axis_index("s") * 2 + lax.axis_index("c")
    per = NP // NW
    base = wid * per

    def _do(src, row0):
        pltpu.sync_copy(src.at[pl.ds(row0, DCH)], xbuf0)
        pltpu.sync_copy(dflat.at[pl.ds(base, DCH)], dbuf0)
        sc0 = pltpu.async_copy(xbuf0, xs_out.at[dbuf0], s0)
        pltpu.sync_copy(src.at[pl.ds(row0 + DCH, DCH)], xbuf1)
        pltpu.sync_copy(dflat.at[pl.ds(base + DCH, DCH)], dbuf1)
        sc1 = pltpu.async_copy(xbuf1, xs_out.at[dbuf1], s1)
        sc0.wait()
        sc1.wait()

    @pl.when(wid < NW // 2)
    def _():
        _do(sx0, base)

    @pl.when(wid >= NW // 2)
    def _():
        _do(sx1, base - N)


def _dispatch(sx0, sx1, dflat):
    fn = functools.partial(
        pl.kernel,
        mesh=plsc.VectorSubcoreMesh(core_axis_name="c", subcore_axis_name="s"),
        out_type=jax.ShapeDtypeStruct((R, D2), _F32),
        scratch_types=[
            pltpu.VMEM((DCH, D2), _F32),
            pltpu.VMEM((DCH, D2), _F32),
            pltpu.VMEM((DCH,), _I32),
            pltpu.VMEM((DCH,), _I32),
            pltpu.SemaphoreType.DMA,
            pltpu.SemaphoreType.DMA,
        ],
    )(_dispatch_body)
    return fn(sx0, sx1, dflat)


def _mm_body(meta_ref, xs_ref, we_ref, yw_ref, wbuf, ybuf, sem_w, sem_y):
    def wcopy(e, slot):
        return pltpu.make_async_copy(we_ref.at[e], wbuf.at[slot], sem_w.at[slot])

    wcopy(0, 0).start()

    def expert(e, g):
        slot = lax.rem(e, 2)

        @pl.when(e + 1 < E)
        def _():
            wcopy(e + 1, lax.rem(e + 1, 2)).start()

        wcopy(e, slot).wait()
        base = meta_ref[0, e] * T
        nt = meta_ref[1, e]

        def tile(i, gi):
            ys = lax.rem(gi, 2)
            row = base + i * T

            def ycopy():
                return pltpu.make_async_copy(
                    ybuf.at[ys], yw_ref.at[pl.ds(row, T)], sem_y.at[ys])

            @pl.when(gi >= 2)
            def _():
                ycopy().wait()

            ybuf[ys, ...] = lax.dot_general(
                _unpack_cols(xs_ref[pl.ds(row, T), :]), wbuf[slot],
                (((1,), (1,)), ((), ())), preferred_element_type=_F32)
            ycopy().start()
            return gi + 1

        return lax.fori_loop(0, nt, tile, g)

    lax.fori_loop(0, E, expert, 0)
    for ys in range(2):
        pltpu.make_async_copy(
            ybuf.at[ys], yw_ref.at[pl.ds(0, T)], sem_y.at[ys]).wait()


def _mm(meta, xs, We, interpret=False):
    return pl.pallas_call(
        _mm_body,
        in_specs=[
            pl.BlockSpec(memory_space=pltpu.SMEM),
            pl.BlockSpec(memory_space=pltpu.VMEM),
            pl.BlockSpec(memory_space=pl.ANY),
        ],
        out_specs=pl.BlockSpec(memory_space=pl.ANY),
        out_shape=jax.ShapeDtypeStruct((R, D), _F32),
        scratch_shapes=[
            pltpu.VMEM((2, D, D), _F32),
            pltpu.VMEM((2, T, D), _F32),
            pltpu.SemaphoreType.DMA((2,)),
            pltpu.SemaphoreType.DMA((2,)),
        ],
        interpret=interpret,
    )(meta, xs, We)


def _combine_body(bias, yw, dflat, out_hbm, acc, g0, g1, db0, db1, s0, s1):
    wid = lax.axis_index("s") * 2 + lax.axis_index("c")
    tpw = N // NW

    def addrows(gbuf):
        def addrow(j, _):
            for k in range(D // 16):
                sl = pl.ds(k * 16, 16)
                acc[j, sl] = acc[j, sl] + gbuf[j, sl]
            return 0

        lax.fori_loop(0, TCH, addrow, 0)

    for c in range(tpw // TCH):
        tb = wid * tpw + c * TCH
        pltpu.sync_copy(dflat.at[pl.ds(tb, TCH)], db0)
        pltpu.sync_copy(dflat.at[pl.ds(N + tb, TCH)], db1)
        cp0 = pltpu.async_copy(yw.at[db0], g0, s0)
        cp1 = pltpu.async_copy(yw.at[db1], g1, s1)
        pltpu.sync_copy(bias.at[pl.ds(tb, TCH)], acc)
        cp0.wait()
        addrows(g0)
        cp1.wait()
        addrows(g1)
        pltpu.sync_copy(acc, out_hbm.at[pl.ds(tb, TCH)])


def _combine(bias, yw, dflat):
    fn = functools.partial(
        pl.kernel,
        mesh=plsc.VectorSubcoreMesh(core_axis_name="c", subcore_axis_name="s"),
        out_type=jax.ShapeDtypeStruct((N, D), _F32),
        scratch_types=[
            pltpu.VMEM((TCH, D), _F32),
            pltpu.VMEM((TCH, D), _F32),
            pltpu.VMEM((TCH, D), _F32),
            pltpu.VMEM((TCH,), _I32),
            pltpu.VMEM((TCH,), _I32),
            pltpu.SemaphoreType.DMA,
            pltpu.SemaphoreType.DMA,
        ],
    )(_combine_body)
    return fn(bias, yw, dflat)


def kernel(x, Wg, bg, We, be):
    bg2 = bg.reshape(1, E)
    idx2, sx0, sx1, bias = _gate(x, Wg, bg2, be)
    dflat2, meta = _route(idx2)
    dflat = dflat2.reshape(NP)
    xs = _dispatch(sx0, sx1, dflat)
    yw = _mm(meta, xs, We)
    return _combine(bias, yw, dflat)


# speedup vs baseline: 1.2812x; 1.0254x over previous
import functools

import jax
import jax.numpy as jnp
from jax import lax
from jax.experimental import pallas as pl
from jax.experimental.pallas import tpu as pltpu
from jax.experimental.pallas import tpu_sc as plsc

N = 4096
D = 768
E = 64
NP = 2 * N
TB = 512
RB = 512
T = 128
NT = 128
R = NT * T
NW = 32
DCH = 128
TCH = 64

_F32 = jnp.float32
_I32 = jnp.int32


D2 = D // 2


def _pack_cols(v):
    b = lax.bitcast_convert_type(v.astype(jnp.bfloat16), jnp.uint16)
    lo = b[:, :D2].astype(jnp.uint32)
    hi = b[:, D2:].astype(jnp.uint32)
    return lax.bitcast_convert_type(lo | (hi << 16), _F32)


def _unpack_cols(p):
    u = lax.bitcast_convert_type(p, jnp.uint32)
    lo = lax.bitcast_convert_type(u << 16, _F32)
    hi = lax.bitcast_convert_type(u & jnp.uint32(0xFFFF0000), _F32)
    return jnp.concatenate([lo, hi], axis=1)


def _gate_body(x_ref, wg_ref, bg_ref, be_ref, i_ref, sx0_ref, sx1_ref, bias_ref):
    x = x_ref[...]
    logits = lax.dot_general(
        x, wg_ref[...], (((1,), (1,)), ((), ())),
        preferred_element_type=_F32) + bg_ref[...]
    m = jnp.max(logits, axis=1, keepdims=True)
    ex = jnp.exp(logits - m)
    p = ex / jnp.sum(ex, axis=1, keepdims=True)
    iota = lax.broadcasted_iota(_I32, (TB, E), 1)
    m1 = jnp.max(p, axis=1, keepdims=True)
    i1 = jnp.min(jnp.where(p == m1, iota, E), axis=1, keepdims=True)
    oh1 = iota == i1
    pm = jnp.where(oh1, -1.0, p)
    m2 = jnp.max(pm, axis=1, keepdims=True)
    i2 = jnp.min(jnp.where(pm == m2, iota, E), axis=1, keepdims=True)
    oh2 = iota == i2
    i_ref[...] = jnp.concatenate([i1, i2], axis=1)
    sx0_ref[...] = _pack_cols(m1 * x)
    sx1_ref[...] = _pack_cols(m2 * x)
    combine = jnp.where(oh1, m1, 0.0) + jnp.where(oh2, m2, 0.0)
    bias_ref[...] = lax.dot_general(
        combine, be_ref[...], (((1,), (0,)), ((), ())),
        preferred_element_type=_F32)


def _gate(x, Wg, bg2, be, interpret=False):
    return pl.pallas_call(
        _gate_body,
        grid=(N // TB,),
        in_specs=[
            pl.BlockSpec((TB, D), lambda i: (i, 0)),
            pl.BlockSpec((E, D), lambda i: (0, 0)),
            pl.BlockSpec((1, E), lambda i: (0, 0)),
            pl.BlockSpec((E, D), lambda i: (0, 0)),
        ],
        out_specs=[
            pl.BlockSpec((TB, 2), lambda i: (i, 0)),
            pl.BlockSpec((TB, D2), lambda i: (i, 0)),
            pl.BlockSpec((TB, D2), lambda i: (i, 0)),
            pl.BlockSpec((TB, D), lambda i: (i, 0)),
        ],
        out_shape=[
            jax.ShapeDtypeStruct((N, 2), _I32),
            jax.ShapeDtypeStruct((N, D2), _F32),
            jax.ShapeDtypeStruct((N, D2), _F32),
            jax.ShapeDtypeStruct((N, D), _F32),
        ],
        interpret=interpret,
    )(x, Wg, bg2, be)


def _route_body(i_ref, dflat_ref, gid_ref, r1_ref, r2_ref):
    nblk = N // RB
    ti = lax.broadcasted_iota(_I32, (RB, RB), 0)
    tj = lax.broadcasted_iota(_I32, (RB, RB), 1)
    tl = (ti > tj).astype(_F32)
    iota_b = lax.broadcasted_iota(_I32, (RB, E), 1)

    def blk(b, carry):
        c0, c1 = carry
        i1b = i_ref[pl.ds(b * RB, RB), pl.ds(0, 1)]
        i2b = i_ref[pl.ds(b * RB, RB), pl.ds(1, 1)]
        h1 = (iota_b == i1b).astype(_F32)
        h2 = (iota_b == i2b).astype(_F32)
        r1_ref[pl.ds(b * RB, RB), :] = lax.dot_general(
            tl, h1, (((1,), (0,)), ((), ())), preferred_element_type=_F32) + c0
        r2_ref[pl.ds(b * RB, RB), :] = lax.dot_general(
            tl, h2, (((1,), (0,)), ((), ())), preferred_element_type=_F32) + c1
        return (c0 + jnp.sum(h1, axis=0, keepdims=True),
                c1 + jnp.sum(h2, axis=0, keepdims=True))

    c0, c1 = lax.fori_loop(
        0, nblk, blk,
        (jnp.zeros((1, E), _F32), jnp.zeros((1, E), _F32)))
    counts = c0 + c1
    ptiles = jnp.floor((counts + float(T - 1)) * (1.0 / T))
    ui = lax.broadcasted_iota(_I32, (E, E), 0)
    uj = lax.broadcasted_iota(_I32, (E, E), 1)
    ul = (ui < uj).astype(_F32)
    tile_off = lax.dot_general(
        ptiles, ul, (((1,), (0,)), ((), ())), preferred_element_type=_F32)
    off_pad = tile_off * float(T)

    iota_f = lax.broadcasted_iota(_I32, (N, E), 1)
    h1f = (iota_f == i_ref[:, pl.ds(0, 1)]).astype(_F32)
    h2f = (iota_f == i_ref[:, pl.ds(1, 1)]).astype(_F32)
    d0 = jnp.sum(h1f * (r1_ref[...] + off_pad), axis=1, keepdims=True)
    d1 = jnp.sum(h2f * (r2_ref[...] + off_pad + c0), axis=1, keepdims=True)
    dflat_ref[pl.ds(0, N), :] = d0.astype(_I32)
    dflat_ref[pl.ds(N, N), :] = d1.astype(_I32)

    gid_ref[pl.ds(0, 1), :] = tile_off.astype(_I32)
    gid_ref[pl.ds(1, 1), :] = ptiles.astype(_I32)


def _route(idx2, interpret=False):
    return pl.pallas_call(
        _route_body,
        out_shape=[
            jax.ShapeDtypeStruct((NP, 1), _I32),
            jax.ShapeDtypeStruct((2, E), _I32),
        ],
        scratch_shapes=[
            pltpu.VMEM((N, E), _F32),
            pltpu.VMEM((N, E), _F32),
        ],
        interpret=interpret,
    )(idx2)


def _dispatch_body(sx0, sx1, dflat, xs_out, xbuf0, xbuf1, dbuf0, dbuf1, s0, s1):
    wid = lax.axis_index("s") * 2 + lax.axis_index("c")
    per = NP // NW
    base = wid * per

    def _do(src, row0):
        pltpu.sync_copy(src.at[pl.ds(row0, DCH)], xbuf0)
        pltpu.sync_copy(dflat.at[pl.ds(base, DCH)], dbuf0)
        sc0 = pltpu.async_copy(xbuf0, xs_out.at[dbuf0], s0)
        pltpu.sync_copy(src.at[pl.ds(row0 + DCH, DCH)], xbuf1)
        pltpu.sync_copy(dflat.at[pl.ds(base + DCH, DCH)], dbuf1)
        sc1 = pltpu.async_copy(xbuf1, xs_out.at[dbuf1], s1)
        sc0.wait()
        sc1.wait()

    @pl.when(wid < NW // 2)
    def _():
        _do(sx0, base)

    @pl.when(wid >= NW // 2)
    def _():
        _do(sx1, base - N)


def _dispatch(sx0, sx1, dflat):
    fn = functools.partial(
        pl.kernel,
        mesh=plsc.VectorSubcoreMesh(core_axis_name="c", subcore_axis_name="s"),
        out_type=jax.ShapeDtypeStruct((R, D2), _F32),
        scratch_types=[
            pltpu.VMEM((DCH, D2), _F32),
            pltpu.VMEM((DCH, D2), _F32),
            pltpu.VMEM((DCH,), _I32),
            pltpu.VMEM((DCH,), _I32),
            pltpu.SemaphoreType.DMA,
            pltpu.SemaphoreType.DMA,
        ],
    )(_dispatch_body)
    return fn(sx0, sx1, dflat)


def _mm_body(meta_ref, xs_ref, we_ref, yw_ref, wbuf, ybuf, sem_w, sem_y):
    def wcopy(e, slot):
        return pltpu.make_async_copy(we_ref.at[e], wbuf.at[slot], sem_w.at[slot])

    wcopy(0, 0).start()

    def expert(e, g):
        slot = lax.rem(e, 2)

        @pl.when(e + 1 < E)
        def _():
            wcopy(e + 1, lax.rem(e + 1, 2)).start()

        wcopy(e, slot).wait()
        base = meta_ref[0, e] * T
        nt = meta_ref[1, e]

        def tile(i, gi):
            ys = lax.rem(gi, 2)
            row = base + i * T

            def ycopy():
                return pltpu.make_async_copy(
                    ybuf.at[ys], yw_ref.at[pl.ds(row, T)], sem_y.at[ys])

            @pl.when(gi >= 2)
            def _():
                ycopy().wait()

            ybuf[ys, ...] = lax.dot_general(
                _unpack_cols(xs_ref[pl.ds(row, T), :]), wbuf[slot],
                (((1,), (1,)), ((), ())), preferred_element_type=_F32)
            ycopy().start()
            return gi + 1

        return lax.fori_loop(0, nt, tile, g)

    lax.fori_loop(0, E, expert, 0)
    for ys in range(2):
        pltpu.make_async_copy(
            ybuf.at[ys], yw_ref.at[pl.ds(0, T)], sem_y.at[ys]).wait()


def _mm(meta, xs, We, interpret=False):
    return pl.pallas_call(
        _mm_body,
        in_specs=[
            pl.BlockSpec(memory_space=pltpu.SMEM),
            pl.BlockSpec(memory_space=pltpu.VMEM),
            pl.BlockSpec(memory_space=pl.ANY),
        ],
        out_specs=pl.BlockSpec(memory_space=pl.ANY),
        out_shape=jax.ShapeDtypeStruct((R, D), _F32),
        scratch_shapes=[
            pltpu.VMEM((2, D, D), _F32),
            pltpu.VMEM((2, T, D), _F32),
            pltpu.SemaphoreType.DMA((2,)),
            pltpu.SemaphoreType.DMA((2,)),
        ],
        interpret=interpret,
    )(meta, xs, We)


def _combine_body(bias, yw, dflat, out_hbm, acc, g0, db0, db1, s0, s1):
    wid = lax.axis_index("s") * 2 + lax.axis_index("c")
    tpw = N // NW

    def addrows(gbuf):
        def addrow(j, _):
            for k in range(D // 16):
                sl = pl.ds(k * 16, 16)
                acc[j, sl] = acc[j, sl] + gbuf[j, sl]
            return 0

        lax.fori_loop(0, TCH, addrow, 0)

    for c in range(tpw // TCH):
        tb = wid * tpw + c * TCH
        pltpu.sync_copy(dflat.at[pl.ds(tb, TCH)], db0)
        pltpu.sync_copy(dflat.at[pl.ds(N + tb, TCH)], db1)
        cp0 = pltpu.async_copy(yw.at[db0], g0, s0)
        pltpu.sync_copy(bias.at[pl.ds(tb, TCH)], acc)
        cp0.wait()
        addrows(g0)
        cp1 = pltpu.async_copy(yw.at[db1], g0, s1)
        cp1.wait()
        addrows(g0)
        pltpu.sync_copy(acc, out_hbm.at[pl.ds(tb, TCH)])


def _combine(bias, yw, dflat):
    fn = functools.partial(
        pl.kernel,
        mesh=plsc.VectorSubcoreMesh(core_axis_name="c", subcore_axis_name="s"),
        out_type=jax.ShapeDtypeStruct((N, D), _F32),
        scratch_types=[
            pltpu.VMEM((TCH, D), _F32),
            pltpu.VMEM((TCH, D), _F32),
            pltpu.VMEM((TCH,), _I32),
            pltpu.VMEM((TCH,), _I32),
            pltpu.SemaphoreType.DMA,
            pltpu.SemaphoreType.DMA,
        ],
    )(_combine_body)
    return fn(bias, yw, dflat)


def kernel(x, Wg, bg, We, be):
    bg2 = bg.reshape(1, E)
    idx2, sx0, sx1, bias = _gate(x, Wg, bg2, be)
    dflat2, meta = _route(idx2)
    dflat = dflat2.reshape(NP)
    xs = _dispatch(sx0, sx1, dflat)
    yw = _mm(meta, xs, We)
    return _combine(bias, yw, dflat)
